# baseline (speedup 1.0000x reference)
import functools

import jax
import jax.numpy as jnp
from jax import lax
from jax.experimental import pallas as pl
from jax.experimental.pallas import tpu as pltpu
from jax.experimental.pallas import tpu_sc as plsc

N = 16384
D = 256
K = 1024
BN = 512
GRID = N // BN


def _dist_argmin_kernel(f_ref, c_ref, fn_ref, cn_ref, lab_ref):
    f = f_ref[...]
    c = c_ref[...]
    mm = lax.dot_general(f, c, (((1,), (1,)), ((), ())),
                         preferred_element_type=jnp.float32)
    fn = fn_ref[...].reshape(BN, 1)
    cn = cn_ref[...].reshape(1, K)
    s = (fn + cn) - 2.0 * mm
    d = jnp.sqrt(jnp.maximum(s, 0.0))
    m = jnp.min(d, axis=1, keepdims=True)
    col = lax.broadcasted_iota(jnp.int32, (BN, K), 1)
    lab = jnp.min(jnp.where(d == m, col, K), axis=1)
    lab_ref[...] = lab.reshape(1, 1, BN)


def _compute_labels(f2d, centroids, fn, cn):
    labs = pl.pallas_call(
        _dist_argmin_kernel,
        grid=(GRID,),
        in_specs=[
            pl.BlockSpec((BN, D), lambda i: (i, 0)),
            pl.BlockSpec((K, D), lambda i: (0, 0)),
            pl.BlockSpec((1, 1, BN), lambda i: (i, 0, 0)),
            pl.BlockSpec((1, K), lambda i: (0, 0)),
        ],
        out_specs=pl.BlockSpec((1, 1, BN), lambda i: (i, 0, 0)),
        out_shape=jax.ShapeDtypeStruct((GRID, 1, BN), jnp.int32),
    )(f2d, centroids, fn.reshape(GRID, 1, BN), cn.reshape(1, K))
    return labs.reshape(N)


_NC, _NS = 2, 16
_NW = _NC * _NS
_BPW = N // _NW
_CH = 128
_NCH = _BPW // _CH

def _sc_gather_body(table_hbm, idx_hbm, out_hbm, idx_v, rows0, rows1, sem0, sem1):
    wid = lax.axis_index("s") * _NC + lax.axis_index("c")
    base = wid * _BPW
    pltpu.sync_copy(idx_hbm.at[pl.ds(wid * _NCH, _NCH)], idx_v)
    bufs = (rows0, rows1)
    sems = (sem0, sem1)
    cps = [None, None]
    for j in range(_NCH):
        b = j % 2
        if cps[b] is not None:
            cps[b].wait()
            pltpu.sync_copy(bufs[b], out_hbm.at[pl.ds(base + (j - 2) * _CH, _CH)])
        cps[b] = pltpu.async_copy(table_hbm.at[idx_v.at[j]], bufs[b], sems[b])
    for j in range(_NCH - 2, _NCH):
        b = j % 2
        cps[b].wait()
        pltpu.sync_copy(bufs[b], out_hbm.at[pl.ds(base + j * _CH, _CH)])


@functools.cache
def _sc_gather():
    mesh = plsc.VectorSubcoreMesh(core_axis_name="c", subcore_axis_name="s",
                                  num_cores=_NC, num_subcores=_NS)
    return pl.kernel(
        _sc_gather_body,
        out_type=jax.ShapeDtypeStruct((N, D), jnp.float32),
        mesh=mesh,
        scratch_types=[
            pltpu.VMEM((_NCH, _CH), jnp.int32),
            pltpu.VMEM((_CH, D), jnp.float32),
            pltpu.VMEM((_CH, D), jnp.float32),
            pltpu.SemaphoreType.DMA,
            pltpu.SemaphoreType.DMA,
        ],
    )


def kernel(feats, centroids):
    batch_shape = feats.shape[:-1]
    f2d = feats.reshape(-1, D)
    fn = jnp.sum(f2d ** 2, axis=-1)
    cn = jnp.sum(centroids ** 2, axis=-1)
    labels = _compute_labels(f2d, centroids, fn, cn)
    assigned = _sc_gather()(centroids, labels.reshape(N // _CH, _CH))
    return labels.reshape(batch_shape), assigned.reshape(*batch_shape, D)


# speedup vs baseline: 1.0306x; 1.0306x over previous
import functools

import jax
import jax.numpy as jnp
from jax import lax
from jax.experimental import pallas as pl
from jax.experimental.pallas import tpu as pltpu
from jax.experimental.pallas import tpu_sc as plsc

N = 16384
D = 256
K = 1024
BN = 512
GRID = N // BN


def _dist_argmin_kernel(f2_ref, c_ref, fn_ref, cn_ref, col_ref, lab_ref):
    f2 = f2_ref[...]
    c = c_ref[...]
    mm2 = lax.dot_general(f2, c, (((1,), (1,)), ((), ())),
                          preferred_element_type=jnp.float32)
    fn = fn_ref[...].reshape(BN, 1)
    cn = cn_ref[...].reshape(1, K)
    s = (fn + cn) + mm2
    xm = jnp.min(s, axis=1, keepdims=True)
    m = jnp.sqrt(jnp.maximum(xm, 0.0))
    mi = lax.bitcast_convert_type(m, jnp.int32)
    nm = lax.bitcast_convert_type(mi + 1, jnp.float32)
    p = m * nm
    pi = lax.bitcast_convert_type(p, jnp.int32)
    lo = lax.bitcast_convert_type(pi - 1, jnp.float32)
    hi = lax.bitcast_convert_type(pi + 1, jnp.float32)
    b = jnp.where(jnp.sqrt(hi) == m, hi,
                  jnp.where(jnp.sqrt(p) == m, p,
                            jnp.where(jnp.sqrt(lo) == m, lo, xm)))
    b = jnp.where(m > 0.0, b, 0.0)
    colf = col_ref[...]
    labf = jnp.min(jnp.where(s <= b, colf, float(K)), axis=1)
    lab_ref[...] = labf.astype(jnp.int32).reshape(1, 1, BN)


def _compute_labels(f2d, centroids, fn, cn):
    labs = pl.pallas_call(
        _dist_argmin_kernel,
        grid=(GRID,),
        in_specs=[
            pl.BlockSpec((BN, D), lambda i: (i, 0)),
            pl.BlockSpec((K, D), lambda i: (0, 0)),
            pl.BlockSpec((1, 1, BN), lambda i: (i, 0, 0)),
            pl.BlockSpec((1, K), lambda i: (0, 0)),
            pl.BlockSpec((1, K), lambda i: (0, 0)),
        ],
        out_specs=pl.BlockSpec((1, 1, BN), lambda i: (i, 0, 0)),
        out_shape=jax.ShapeDtypeStruct((GRID, 1, BN), jnp.int32),
    )(f2d, centroids, fn.reshape(GRID, 1, BN), cn.reshape(1, K),
      jnp.arange(K, dtype=jnp.float32).reshape(1, K))
    return labs.reshape(N)


_NC, _NS = 2, 16
_NW = _NC * _NS
_BPW = N // _NW
_CH = 128
_NCH = _BPW // _CH

def _sc_gather_body(table_hbm, idx_hbm, out_hbm, idx_v, rows0, rows1, sem0, sem1):
    wid = lax.axis_index("s") * _NC + lax.axis_index("c")
    base = wid * _BPW
    pltpu.sync_copy(idx_hbm.at[pl.ds(wid * _NCH, _NCH)], idx_v)
    bufs = (rows0, rows1)
    sems = (sem0, sem1)
    cps = [None, None]
    for j in range(_NCH):
        b = j % 2
        if cps[b] is not None:
            cps[b].wait()
            pltpu.sync_copy(bufs[b], out_hbm.at[pl.ds(base + (j - 2) * _CH, _CH)])
        cps[b] = pltpu.async_copy(table_hbm.at[idx_v.at[j]], bufs[b], sems[b])
    for j in range(_NCH - 2, _NCH):
        b = j % 2
        cps[b].wait()
        pltpu.sync_copy(bufs[b], out_hbm.at[pl.ds(base + j * _CH, _CH)])


@functools.cache
def _sc_gather():
    mesh = plsc.VectorSubcoreMesh(core_axis_name="c", subcore_axis_name="s",
                                  num_cores=_NC, num_subcores=_NS)
    return pl.kernel(
        _sc_gather_body,
        out_type=jax.ShapeDtypeStruct((N, D), jnp.float32),
        mesh=mesh,
        scratch_types=[
            pltpu.VMEM((_NCH, _CH), jnp.int32),
            pltpu.VMEM((_CH, D), jnp.float32),
            pltpu.VMEM((_CH, D), jnp.float32),
            pltpu.SemaphoreType.DMA,
            pltpu.SemaphoreType.DMA,
        ],
    )


def kernel(feats, centroids):
    batch_shape = feats.shape[:-1]
    f2d = feats.reshape(-1, D)
    fn = jnp.sum(f2d ** 2, axis=-1)
    cn = jnp.sum(centroids ** 2, axis=-1)
    labels = _compute_labels(f2d * -2.0, centroids, fn, cn)
    assigned = _sc_gather()(centroids, labels.reshape(N // _CH, _CH))
    return labels.reshape(batch_shape), assigned.reshape(*batch_shape, D)


# speedup vs baseline: 1.0583x; 1.0268x over previous
import functools

import jax
import jax.numpy as jnp
from jax import lax
from jax.experimental import pallas as pl
from jax.experimental.pallas import tpu as pltpu
from jax.experimental.pallas import tpu_sc as plsc

N = 16384
D = 256
K = 1024
BN = 512
GRID = N // BN


def _dist_argmin_kernel(f2_ref, c_ref, fn_ref, cn_ref, col_ref, lab_ref):
    f2 = f2_ref[...] * -2.0
    c = c_ref[...]
    mm2 = lax.dot_general(f2, c, (((1,), (1,)), ((), ())),
                          preferred_element_type=jnp.float32)
    fn = fn_ref[...].reshape(BN, 1)
    cn = cn_ref[...].reshape(1, K)
    s = (fn + cn) + mm2
    xm = jnp.min(s, axis=1, keepdims=True)
    m = jnp.sqrt(jnp.maximum(xm, 0.0))
    mi = lax.bitcast_convert_type(m, jnp.int32)
    nm = lax.bitcast_convert_type(mi + 1, jnp.float32)
    p = m * nm
    pi = lax.bitcast_convert_type(p, jnp.int32)
    lo = lax.bitcast_convert_type(pi - 1, jnp.float32)
    hi = lax.bitcast_convert_type(pi + 1, jnp.float32)
    b = jnp.where(jnp.sqrt(hi) == m, hi,
                  jnp.where(jnp.sqrt(p) == m, p,
                            jnp.where(jnp.sqrt(lo) == m, lo, xm)))
    b = jnp.where(m > 0.0, b, 0.0)
    colf = col_ref[...]
    labf = jnp.min(jnp.where(s <= b, colf, float(K)), axis=1)
    lab_ref[...] = labf.astype(jnp.int32).reshape(1, 1, BN)


def _compute_labels(f2d, centroids, fn, cn):
    labs = pl.pallas_call(
        _dist_argmin_kernel,
        grid=(GRID,),
        in_specs=[
            pl.BlockSpec((BN, D), lambda i: (i, 0)),
            pl.BlockSpec((K, D), lambda i: (0, 0)),
            pl.BlockSpec((1, 1, BN), lambda i: (i, 0, 0)),
            pl.BlockSpec((1, K), lambda i: (0, 0)),
            pl.BlockSpec((1, K), lambda i: (0, 0)),
        ],
        out_specs=pl.BlockSpec((1, 1, BN), lambda i: (i, 0, 0)),
        out_shape=jax.ShapeDtypeStruct((GRID, 1, BN), jnp.int32),
    )(f2d, centroids, fn.reshape(GRID, 1, BN), cn.reshape(1, K),
      jnp.arange(K, dtype=jnp.float32).reshape(1, K))
    return labs.reshape(N)


_NC, _NS = 2, 16
_NW = _NC * _NS
_BPW = N // _NW
_CH = 128
_NCH = _BPW // _CH

def _sc_gather_body(table_hbm, idx_hbm, out_hbm, idx_v,
                    rows0, rows1, gs0, gs1, os0, os1):
    wid = lax.axis_index("s") * _NC + lax.axis_index("c")
    base = wid * _BPW
    pltpu.sync_copy(idx_hbm.at[pl.ds(wid * _NCH, _NCH)], idx_v)
    bufs = (rows0, rows1)
    gsems = (gs0, gs1)
    osems = (os0, os1)
    gcp = [None] * _NCH
    ocp = [None] * _NCH
    for j in range(_NCH):
        b = j % 2
        if j >= 2:
            ocp[j - 2].wait()
        gcp[j] = pltpu.async_copy(table_hbm.at[idx_v.at[j]], bufs[b], gsems[b])
        if j >= 1:
            pb = (j - 1) % 2
            gcp[j - 1].wait()
            ocp[j - 1] = pltpu.async_copy(
                bufs[pb], out_hbm.at[pl.ds(base + (j - 1) * _CH, _CH)], osems[pb])
    lb = (_NCH - 1) % 2
    gcp[_NCH - 1].wait()
    ocp[_NCH - 1] = pltpu.async_copy(
        bufs[lb], out_hbm.at[pl.ds(base + (_NCH - 1) * _CH, _CH)], osems[lb])
    ocp[_NCH - 2].wait()
    ocp[_NCH - 1].wait()


@functools.cache
def _sc_gather():
    mesh = plsc.VectorSubcoreMesh(core_axis_name="c", subcore_axis_name="s",
                                  num_cores=_NC, num_subcores=_NS)
    return pl.kernel(
        _sc_gather_body,
        out_type=jax.ShapeDtypeStruct((N, D), jnp.float32),
        mesh=mesh,
        scratch_types=[
            pltpu.VMEM((_NCH, _CH), jnp.int32),
            pltpu.VMEM((_CH, D), jnp.float32),
            pltpu.VMEM((_CH, D), jnp.float32),
            pltpu.SemaphoreType.DMA,
            pltpu.SemaphoreType.DMA,
            pltpu.SemaphoreType.DMA,
            pltpu.SemaphoreType.DMA,
        ],
    )


def kernel(feats, centroids):
    batch_shape = feats.shape[:-1]
    f2d = feats.reshape(-1, D)
    fn = jnp.sum(f2d ** 2, axis=-1)
    cn = jnp.sum(centroids ** 2, axis=-1)
    labels = _compute_labels(f2d, centroids, fn, cn)
    assigned = _sc_gather()(centroids, labels.reshape(N // _CH, _CH))
    return labels.reshape(batch_shape), assigned.reshape(*batch_shape, D)
